# baseline (speedup 1.0000x reference)
import jax
import jax.numpy as jnp
from jax.experimental import pallas as pl
from jax.experimental.pallas import tpu as pltpu

_K = 1024
_D = 64
_L = 8
_M = 512
_G = 4
_KG = _K // _G
_GD = _G * _D


def _rvq_body(x_ref, cb_ref, cbg_ref, out_ref, idx_ref, c2_ref, cbs_ref,
              cbb_ref):
    b = pl.program_id(0)
    t = pl.program_id(1)

    @pl.when((b == 0) & (t == 0))
    def _prep():
        for l in range(_L):
            cb = cb_ref[l]
            c2_ref[l, :] = jnp.sum(cb * cb, axis=1)
            cbb_ref[l, :, :] = cb.astype(jnp.bfloat16)
            rem = cbg_ref[l]
            for p in range(3):
                plane = rem.astype(jnp.bfloat16)
                cbs_ref[l, p, :, :] = plane
                rem = rem - plane.astype(jnp.float32)

    x = x_ref[0].T
    r = x
    out = jnp.zeros_like(x)
    i128 = jax.lax.broadcasted_iota(jnp.int32, (_M, 128), 1)
    for l in range(_L):
        s = jax.lax.dot_general(
            r.astype(jnp.bfloat16), cbb_ref[l],
            (((1,), (1,)), ((), ())),
            preferred_element_type=jnp.float32)
        r2 = jnp.sum(r * r, axis=1, keepdims=True)
        d2 = r2 - 2.0 * s + c2_ref[l, :][None, :]

        mc = d2[:, 0:128]
        for c in range(1, 8):
            mc = jnp.minimum(mc, d2[:, 128 * c:128 * (c + 1)])
        for sh in (64, 32, 16, 8, 4, 2, 1):
            mc = jnp.minimum(mc, jnp.roll(mc, sh, axis=1))
        cand = None
        for c in range(8):
            eq = d2[:, 128 * c:128 * (c + 1)] == mc
            cc = jnp.where(eq, i128 + 128 * c, _K)
            cand = cc if cand is None else jnp.minimum(cand, cc)
        for sh in (64, 32, 16, 8, 4, 2, 1):
            cand = jnp.minimum(cand, jnp.roll(cand, sh, axis=1))

        g = cand // _G
        s4 = cand % _G
        ohA = (g == i128).astype(jnp.bfloat16)
        ohB = (g == i128 + 128).astype(jnp.bfloat16)
        oh = jnp.concatenate([ohA, ohB, ohA, ohB, ohA, ohB], axis=1)
        planes = cbs_ref[l].reshape(3 * _KG, _GD)
        s1 = jax.lax.dot_general(
            oh, planes, (((1,), (0,)), ((), ())),
            preferred_element_type=jnp.float32)
        pat0 = i128 // _D
        selA = (s4 == pat0).astype(jnp.float32)
        selB = (s4 == pat0 + 2).astype(jnp.float32)
        sel = jnp.concatenate([selA, selB], axis=1)
        picked = s1 * sel
        q = (picked[:, 0 * _D:1 * _D] + picked[:, 1 * _D:2 * _D]
             + picked[:, 2 * _D:3 * _D] + picked[:, 3 * _D:4 * _D])
        out = out + q
        r = r - q
        idx_ref[0, l, :] = cand[:, 0]
    out_ref[0] = out.T


def kernel(input, codebooks):
    B, D, T = input.shape
    L = codebooks.shape[0]
    cbg = codebooks.reshape(L, _KG, _GD)

    out, idx = pl.pallas_call(
        _rvq_body,
        grid=(B, T // _M),
        in_specs=[
            pl.BlockSpec((1, D, _M), lambda b, t: (b, 0, t)),
            pl.BlockSpec((L, _K, D), lambda b, t: (0, 0, 0)),
            pl.BlockSpec((L, _KG, _GD), lambda b, t: (0, 0, 0)),
        ],
        out_specs=[
            pl.BlockSpec((1, D, _M), lambda b, t: (b, 0, t)),
            pl.BlockSpec((1, L, _M), lambda b, t: (b, 0, t)),
        ],
        out_shape=[
            jax.ShapeDtypeStruct((B, D, T), jnp.float32),
            jax.ShapeDtypeStruct((B, L, T), jnp.int32),
        ],
        scratch_shapes=[
            pltpu.VMEM((_L, _K), jnp.float32),
            pltpu.VMEM((_L, 3, _KG, _GD), jnp.bfloat16),
            pltpu.VMEM((_L, _K, _D), jnp.bfloat16),
        ],
    )(input, codebooks, cbg)

    return out, idx.astype(jnp.int64)


# speedup vs baseline: 1.0010x; 1.0010x over previous
import jax
import jax.numpy as jnp
from jax.experimental import pallas as pl
from jax.experimental.pallas import tpu as pltpu

_K = 1024
_D = 64
_L = 8
_M = 512
_G = 4
_KG = _K // _G
_GD = _G * _D


def _rvq_body(x_ref, cb_ref, cbg_ref, out_ref, idx_ref, c2_ref, cbs_ref,
              cbb_ref):
    b = pl.program_id(0)
    t = pl.program_id(1)

    @pl.when((b == 0) & (t == 0))
    def _prep():
        for l in range(_L):
            cb = cb_ref[l]
            c2_ref[l, :] = jnp.sum(cb * cb, axis=1)
            cbb_ref[l, :, :] = cb.astype(jnp.bfloat16)
            rem = cbg_ref[l]
            for p in range(3):
                plane = rem.astype(jnp.bfloat16)
                cbs_ref[l, p, :, :] = plane
                rem = rem - plane.astype(jnp.float32)

    x = x_ref[0].T
    r = x
    out = jnp.zeros_like(x)
    i128 = jax.lax.broadcasted_iota(jnp.int32, (_M, 128), 1)
    for l in range(_L):
        s = jax.lax.dot_general(
            r.astype(jnp.bfloat16), cbb_ref[l],
            (((1,), (1,)), ((), ())),
            preferred_element_type=jnp.float32)
        r2 = jnp.sum(r * r, axis=1, keepdims=True)
        d2 = r2 - 2.0 * s + c2_ref[l, :][None, :]

        mc = d2[:, 0:128]
        for c in range(1, 8):
            mc = jnp.minimum(mc, d2[:, 128 * c:128 * (c + 1)])
        for sh in (64, 32, 16, 8, 4, 2, 1):
            mc = jnp.minimum(mc, pltpu.roll(mc, sh, 1))
        cand = None
        for c in range(8):
            eq = d2[:, 128 * c:128 * (c + 1)] == mc
            cc = jnp.where(eq, i128 + 128 * c, _K)
            cand = cc if cand is None else jnp.minimum(cand, cc)
        for sh in (64, 32, 16, 8, 4, 2, 1):
            cand = jnp.minimum(cand, pltpu.roll(cand, sh, 1))

        g = cand // _G
        s4 = cand % _G
        ohA = (g == i128).astype(jnp.bfloat16)
        ohB = (g == i128 + 128).astype(jnp.bfloat16)
        oh = jnp.concatenate([ohA, ohB, ohA, ohB, ohA, ohB], axis=1)
        planes = cbs_ref[l].reshape(3 * _KG, _GD)
        s1 = jax.lax.dot_general(
            oh, planes, (((1,), (0,)), ((), ())),
            preferred_element_type=jnp.float32)
        pat0 = i128 // _D
        selA = (s4 == pat0).astype(jnp.float32)
        selB = (s4 == pat0 + 2).astype(jnp.float32)
        sel = jnp.concatenate([selA, selB], axis=1)
        picked = s1 * sel
        q = (picked[:, 0 * _D:1 * _D] + picked[:, 1 * _D:2 * _D]
             + picked[:, 2 * _D:3 * _D] + picked[:, 3 * _D:4 * _D])
        out = out + q
        r = r - q
        idx_ref[0, l, :] = cand[:, 0]
    out_ref[0] = out.T


def kernel(input, codebooks):
    B, D, T = input.shape
    L = codebooks.shape[0]
    cbg = codebooks.reshape(L, _KG, _GD)

    out, idx = pl.pallas_call(
        _rvq_body,
        grid=(B, T // _M),
        in_specs=[
            pl.BlockSpec((1, D, _M), lambda b, t: (b, 0, t)),
            pl.BlockSpec((L, _K, D), lambda b, t: (0, 0, 0)),
            pl.BlockSpec((L, _KG, _GD), lambda b, t: (0, 0, 0)),
        ],
        out_specs=[
            pl.BlockSpec((1, D, _M), lambda b, t: (b, 0, t)),
            pl.BlockSpec((1, L, _M), lambda b, t: (b, 0, t)),
        ],
        out_shape=[
            jax.ShapeDtypeStruct((B, D, T), jnp.float32),
            jax.ShapeDtypeStruct((B, L, T), jnp.int32),
        ],
        scratch_shapes=[
            pltpu.VMEM((_L, _K), jnp.float32),
            pltpu.VMEM((_L, 3, _KG, _GD), jnp.bfloat16),
            pltpu.VMEM((_L, _K, _D), jnp.bfloat16),
        ],
    )(input, codebooks, cbg)

    return out, idx.astype(jnp.int64)


# speedup vs baseline: 1.6769x; 1.6752x over previous
import jax
import jax.numpy as jnp
from jax.experimental import pallas as pl
from jax.experimental.pallas import tpu as pltpu

_K = 1024
_D = 64
_L = 8
_M = 512
_G = 4
_KG = _K // _G
_GD = _G * _D


def _rvq_body(x_ref, cb_ref, cbg_ref, out_ref, idx_ref, c2_ref, cbs_ref,
              cbb_ref):
    b = pl.program_id(0)
    t = pl.program_id(1)

    @pl.when((b == 0) & (t == 0))
    def _prep():
        for l in range(_L):
            cb = cb_ref[l]
            c2_ref[l, :] = jnp.sum(cb * cb, axis=1)
            cbb_ref[l, :, :] = cb.astype(jnp.bfloat16)
            rem = cbg_ref[l]
            for p in range(3):
                plane = rem.astype(jnp.bfloat16)
                cbs_ref[l, p, :, :] = plane
                rem = rem - plane.astype(jnp.float32)

    x = x_ref[0].T
    r = x
    out = jnp.zeros_like(x)
    iota_k = jax.lax.broadcasted_iota(jnp.int32, (_M, _K), 1)
    lane_g = jax.lax.broadcasted_iota(jnp.int32, (_M, 3 * _KG), 1) % _KG
    lane_s = jax.lax.broadcasted_iota(jnp.int32, (_M, _GD), 1) // _D
    for l in range(_L):
        s = jax.lax.dot_general(
            r.astype(jnp.bfloat16), cbb_ref[l],
            (((1,), (1,)), ((), ())),
            preferred_element_type=jnp.float32)
        r2 = jnp.sum(r * r, axis=1, keepdims=True)
        d2 = r2 - 2.0 * s + c2_ref[l, :][None, :]

        m = jnp.min(d2, axis=1, keepdims=True)
        cand = jnp.where(d2 == m, iota_k, _K)
        idx = jnp.min(cand, axis=1)

        oh = (lane_g == (idx // _G)[:, None]).astype(jnp.bfloat16)
        planes = cbs_ref[l].reshape(3 * _KG, _GD)
        s1 = jax.lax.dot_general(
            oh, planes, (((1,), (0,)), ((), ())),
            preferred_element_type=jnp.float32)
        sel = (lane_s == (idx % _G)[:, None]).astype(jnp.float32)
        picked = s1 * sel
        q = (picked[:, 0 * _D:1 * _D] + picked[:, 1 * _D:2 * _D]
             + picked[:, 2 * _D:3 * _D] + picked[:, 3 * _D:4 * _D])
        out = out + q
        r = r - q
        idx_ref[0, l, :] = idx
    out_ref[0] = out.T


def kernel(input, codebooks):
    B, D, T = input.shape
    L = codebooks.shape[0]
    cbg = codebooks.reshape(L, _KG, _GD)

    out, idx = pl.pallas_call(
        _rvq_body,
        grid=(B, T // _M),
        in_specs=[
            pl.BlockSpec((1, D, _M), lambda b, t: (b, 0, t)),
            pl.BlockSpec((L, _K, D), lambda b, t: (0, 0, 0)),
            pl.BlockSpec((L, _KG, _GD), lambda b, t: (0, 0, 0)),
        ],
        out_specs=[
            pl.BlockSpec((1, D, _M), lambda b, t: (b, 0, t)),
            pl.BlockSpec((1, L, _M), lambda b, t: (b, 0, t)),
        ],
        out_shape=[
            jax.ShapeDtypeStruct((B, D, T), jnp.float32),
            jax.ShapeDtypeStruct((B, L, T), jnp.int32),
        ],
        scratch_shapes=[
            pltpu.VMEM((_L, _K), jnp.float32),
            pltpu.VMEM((_L, 3, _KG, _GD), jnp.bfloat16),
            pltpu.VMEM((_L, _K, _D), jnp.bfloat16),
        ],
    )(input, codebooks, cbg)

    return out, idx.astype(jnp.int64)
